# baseline (speedup 1.0000x reference)
import functools

import jax
import jax.numpy as jnp
from jax import lax
from jax.experimental import pallas as pl
from jax.experimental.pallas import tpu as pltpu
from jax.experimental.pallas import tpu_sc as plsc

B = 16384
EMB = 32

_f32 = jnp.float32


@functools.lru_cache(maxsize=None)
def _sc_gather_fn():
    info = plsc.get_sparse_core_info()
    nc, ns = info.num_cores, info.num_subcores
    nw = nc * ns
    bpw = B // nw
    mesh = plsc.VectorSubcoreMesh(
        core_axis_name="c", subcore_axis_name="s", num_cores=nc,
        num_subcores=ns,
    )

    def body(
        cust_i, art_i, age_i, idxg_i, gar_i,
        cust_t, art_t, age_t, idxg_t, gar_t, cb_t, ab_t,
        cust_o, art_o, age_o, idxg_o, gar_o, cb_o, ab_o,
        cust_iv, art_iv, age_iv, idxg_iv, gar_iv,
        cust_v, art_v, age_v, idxg_v, gar_v, cb_v, ab_v,
        sem,
    ):
        wid = lax.axis_index("s") * nc + lax.axis_index("c")
        sl = pl.ds(wid * bpw, bpw)

        pltpu.sync_copy(cust_i.at[sl], cust_iv)
        pltpu.sync_copy(art_i.at[sl], art_iv)
        pltpu.sync_copy(age_i.at[sl], age_iv)
        pltpu.sync_copy(idxg_i.at[sl], idxg_iv)
        pltpu.sync_copy(gar_i.at[sl], gar_iv)

        c0 = pltpu.async_copy(cust_t.at[cust_iv], cust_v, sem)
        c1 = pltpu.async_copy(art_t.at[art_iv], art_v, sem)
        c2 = pltpu.async_copy(age_t.at[age_iv], age_v, sem)
        c3 = pltpu.async_copy(idxg_t.at[idxg_iv], idxg_v, sem)
        c4 = pltpu.async_copy(gar_t.at[gar_iv], gar_v, sem)
        c5 = pltpu.async_copy(cb_t.at[cust_iv], cb_v, sem)
        c6 = pltpu.async_copy(ab_t.at[art_iv], ab_v, sem)
        for c in (c0, c1, c2, c3, c4, c5, c6):
            c.wait()

        pltpu.sync_copy(cust_v, cust_o.at[sl])
        pltpu.sync_copy(art_v, art_o.at[sl])
        pltpu.sync_copy(age_v, age_o.at[sl])
        pltpu.sync_copy(idxg_v, idxg_o.at[sl])
        pltpu.sync_copy(gar_v, gar_o.at[sl])
        pltpu.sync_copy(cb_v, cb_o.at[sl])
        pltpu.sync_copy(ab_v, ab_o.at[sl])

    return pl.kernel(
        body,
        out_type=(
            jax.ShapeDtypeStruct((B, EMB), _f32),
            jax.ShapeDtypeStruct((B, EMB), _f32),
            jax.ShapeDtypeStruct((B, EMB), _f32),
            jax.ShapeDtypeStruct((B, EMB), _f32),
            jax.ShapeDtypeStruct((B, EMB), _f32),
            jax.ShapeDtypeStruct((B, 1), _f32),
            jax.ShapeDtypeStruct((B, 1), _f32),
        ),
        mesh=mesh,
        compiler_params=pltpu.CompilerParams(use_tc_tiling_on_sc=False),
        scratch_types=[
            pltpu.VMEM((bpw,), jnp.int32),
            pltpu.VMEM((bpw,), jnp.int32),
            pltpu.VMEM((bpw,), jnp.int32),
            pltpu.VMEM((bpw,), jnp.int32),
            pltpu.VMEM((bpw,), jnp.int32),
            pltpu.VMEM((bpw, EMB), _f32),
            pltpu.VMEM((bpw, EMB), _f32),
            pltpu.VMEM((bpw, EMB), _f32),
            pltpu.VMEM((bpw, EMB), _f32),
            pltpu.VMEM((bpw, EMB), _f32),
            pltpu.VMEM((bpw, 1), _f32),
            pltpu.VMEM((bpw, 1), _f32),
            pltpu.SemaphoreType.DMA,
        ],
    )


def _sigmoid(x):
    return 1.0 / (1.0 + jnp.exp(-x))


TB = 2048


def _tc_body(cust_r, age_r, art_r, idxg_r, gar_r, cb_r, ab_r,
             wc_r, bc_r, wa_r, ba_r, out_r):
    wc = wc_r[...]
    wa = wa_r[...]
    cm = _sigmoid(
        jnp.dot(cust_r[...], wc[0:EMB], preferred_element_type=_f32)
        + jnp.dot(age_r[...], wc[EMB:2 * EMB], preferred_element_type=_f32)
        + bc_r[...]
    )
    am = _sigmoid(
        jnp.dot(art_r[...], wa[0:EMB], preferred_element_type=_f32)
        + jnp.dot(idxg_r[...], wa[EMB:2 * EMB], preferred_element_type=_f32)
        + jnp.dot(gar_r[...], wa[2 * EMB:3 * EMB], preferred_element_type=_f32)
        + ba_r[...]
    )
    x = jnp.sum(cm * am, axis=1, keepdims=True) + cb_r[...] + ab_r[...]
    out_r[...] = _sigmoid(x)


def _tc_dense(cust_r, age_r, art_r, idxg_r, gar_r, cb, ab, w_cust, b_cust,
              w_art, b_art):
    n_blocks = B // TB
    row_spec = pl.BlockSpec((TB, EMB), lambda i: (i, 0))
    col_spec = pl.BlockSpec((TB, 1), lambda i: (i, 0))
    return pl.pallas_call(
        _tc_body,
        grid=(n_blocks,),
        in_specs=[
            row_spec, row_spec, row_spec, row_spec, row_spec,
            col_spec, col_spec,
            pl.BlockSpec((2 * EMB, EMB), lambda i: (0, 0)),
            pl.BlockSpec((1, EMB), lambda i: (0, 0)),
            pl.BlockSpec((3 * EMB, EMB), lambda i: (0, 0)),
            pl.BlockSpec((1, EMB), lambda i: (0, 0)),
        ],
        out_specs=col_spec,
        out_shape=jax.ShapeDtypeStruct((B, 1), _f32),
    )(cust_r, age_r, art_r, idxg_r, gar_r, cb, ab, w_cust, b_cust,
      w_art, b_art)


def kernel(row, customer_embed, art_embed, customer_bias, article_bias,
           age_embed, indexgroup_embed, garmentgroup_embed,
           W_art, b_art, W_cust, b_cust):
    row = row.astype(jnp.int32)
    cust = row[:, 0]
    art = row[:, 1]
    age = jnp.where(row[:, 2] < 0, 36, row[:, 2]) - 1
    gar = row[:, 3] - 1001
    idxg = row[:, 4] - 1

    cust_r, art_r, age_r, idxg_r, gar_r, cb, ab = _sc_gather_fn()(
        cust, art, age, idxg, gar,
        customer_embed, art_embed, age_embed, indexgroup_embed,
        garmentgroup_embed, customer_bias, article_bias,
    )
    return _tc_dense(
        cust_r, age_r, art_r, idxg_r, gar_r, cb, ab,
        W_cust, b_cust.reshape(1, EMB), W_art, b_art.reshape(1, EMB),
    )


# speedup vs baseline: 2.5444x; 2.5444x over previous
import functools

import jax
import jax.numpy as jnp
from jax import lax
from jax.experimental import pallas as pl
from jax.experimental.pallas import tpu as pltpu
from jax.experimental.pallas import tpu_sc as plsc

B = 16384
EMB = 32

_f32 = jnp.float32


@functools.lru_cache(maxsize=None)
def _sc_gather_fn():
    info = plsc.get_sparse_core_info()
    nc, ns = info.num_cores, info.num_subcores
    nw = nc * ns
    bpw = B // nw

    mesh = plsc.VectorSubcoreMesh(
        core_axis_name="c", subcore_axis_name="s", num_cores=nc,
        num_subcores=ns,
    )

    def body(cust_i, art_i, cust_t, art_t, cb_t, ab_t,
             cust_o, art_o, cb_o, ab_o,
             cidx_v, aidx_v, bufc, bufa, cbv, abv, sem):
        wid = lax.axis_index("s") * nc + lax.axis_index("c")
        sl = pl.ds(wid * bpw, bpw)
        pltpu.sync_copy(cust_i.at[sl], cidx_v)
        pltpu.sync_copy(art_i.at[sl], aidx_v)

        c0 = pltpu.async_copy(cust_t.at[cidx_v], bufc, sem)
        c1 = pltpu.async_copy(art_t.at[aidx_v], bufa, sem)
        c2 = pltpu.async_copy(cb_t.at[cidx_v], cbv, sem)
        c3 = pltpu.async_copy(ab_t.at[aidx_v], abv, sem)
        for c in (c0, c1, c2, c3):
            c.wait()

        pltpu.sync_copy(bufc, cust_o.at[sl])
        pltpu.sync_copy(bufa, art_o.at[sl])
        pltpu.sync_copy(cbv, cb_o.at[sl])
        pltpu.sync_copy(abv, ab_o.at[sl])

    return pl.kernel(
        body,
        out_type=(
            jax.ShapeDtypeStruct((B, EMB), _f32),
            jax.ShapeDtypeStruct((B, EMB), _f32),
            jax.ShapeDtypeStruct((B,), _f32),
            jax.ShapeDtypeStruct((B,), _f32),
        ),
        mesh=mesh,
        compiler_params=pltpu.CompilerParams(use_tc_tiling_on_sc=False),
        scratch_types=[
            pltpu.VMEM((bpw,), jnp.int32),
            pltpu.VMEM((bpw,), jnp.int32),
            pltpu.VMEM((bpw, EMB), _f32),
            pltpu.VMEM((bpw, EMB), _f32),
            pltpu.VMEM((bpw,), _f32),
            pltpu.VMEM((bpw,), _f32),
            pltpu.SemaphoreType.DMA,
        ],
    )


def _sigmoid(x):
    return 1.0 / (1.0 + jnp.exp(-x))


TB = 2048

NUM_AGE = 100
NUM_IDXGROUP = 10
NUM_GARMENT = 21


def _onehot(idx_blk, n):
    classes = lax.broadcasted_iota(jnp.int32, (idx_blk.shape[0], n), 1)
    return jnp.where(classes == idx_blk, 1.0, 0.0).astype(_f32)


def _dot(a, b):
    return jnp.dot(a, b, preferred_element_type=_f32)


def _tc_body(cust_r, art_r, cb_r, ab_r, age_r, idxg_r, gar_r,
             age_t, idxg_t, gar_t, wc_r, bc_r, wa_r, ba_r, out_r):
    wc = wc_r[...]
    wa = wa_r[...]
    age_proj = _dot(age_t[...], wc[EMB:2 * EMB])
    idxg_proj = _dot(idxg_t[...], wa[EMB:2 * EMB])
    gar_proj = _dot(gar_t[...], wa[2 * EMB:3 * EMB])

    cm = _sigmoid(
        _dot(cust_r[...], wc[0:EMB])
        + _dot(_onehot(age_r[...], NUM_AGE), age_proj)
        + bc_r[...]
    )
    am = _sigmoid(
        _dot(art_r[...], wa[0:EMB])
        + _dot(_onehot(idxg_r[...], NUM_IDXGROUP), idxg_proj)
        + _dot(_onehot(gar_r[...], NUM_GARMENT), gar_proj)
        + ba_r[...]
    )
    x = jnp.sum(cm * am, axis=1, keepdims=True) + cb_r[...] + ab_r[...]
    out_r[...] = _sigmoid(x)


def _tc_dense(cust_rows, art_rows, cb, ab, age_i, idxg_i, gar_i,
              age_t, idxg_t, gar_t, w_cust, b_cust, w_art, b_art):
    n_blocks = B // TB
    row_spec = pl.BlockSpec((TB, EMB), lambda i: (i, 0))
    col_spec = pl.BlockSpec((TB, 1), lambda i: (i, 0))
    full = lambda shape: pl.BlockSpec(shape, lambda i: (0, 0))
    return pl.pallas_call(
        _tc_body,
        grid=(n_blocks,),
        in_specs=[
            row_spec, row_spec, col_spec, col_spec,
            col_spec, col_spec, col_spec,
            full((NUM_AGE, EMB)), full((NUM_IDXGROUP, EMB)),
            full((NUM_GARMENT, EMB)),
            full((2 * EMB, EMB)), full((1, EMB)),
            full((3 * EMB, EMB)), full((1, EMB)),
        ],
        out_specs=col_spec,
        out_shape=jax.ShapeDtypeStruct((B, 1), _f32),
    )(cust_rows, art_rows, cb, ab, age_i, idxg_i, gar_i,
      age_t, idxg_t, gar_t, w_cust, b_cust, w_art, b_art)


def kernel(row, customer_embed, art_embed, customer_bias, article_bias,
           age_embed, indexgroup_embed, garmentgroup_embed,
           W_art, b_art, W_cust, b_cust):
    row = row.astype(jnp.int32)
    cust = row[:, 0]
    art = row[:, 1]
    age = jnp.where(row[:, 2] < 0, 36, row[:, 2]) - 1
    gar = row[:, 3] - 1001
    idxg = row[:, 4] - 1

    cust_rows, art_rows, cb, ab = _sc_gather_fn()(
        cust, art, customer_embed, art_embed,
        customer_bias.reshape(-1), article_bias.reshape(-1),
    )
    return _tc_dense(
        cust_rows, art_rows, cb.reshape(B, 1), ab.reshape(B, 1),
        age.reshape(B, 1), idxg.reshape(B, 1), gar.reshape(B, 1),
        age_embed, indexgroup_embed, garmentgroup_embed,
        W_cust, b_cust.reshape(1, EMB), W_art, b_art.reshape(1, EMB),
    )
